# Initial kernel scaffold; baseline (speedup 1.0000x reference)
import jax
import jax.numpy as jnp
from jax.experimental import pallas as pl


def kernel(position_ids, pos_embed):
    raise NotImplementedError("write your pallas kernel here")


# speedup vs baseline: 2.2786x; 2.2786x over previous
import functools

import jax
import jax.numpy as jnp
from jax import lax
from jax.experimental import pallas as pl
from jax.experimental.pallas import tpu as pltpu
from jax.experimental.pallas import tpu_sc as plsc

_B = 64 * 1024
_D = 768
_C = 128


@functools.cache
def _build_gather():
    info = plsc.get_sparse_core_info()
    num_cores, num_subcores = info.num_cores, info.num_subcores
    num_workers = num_cores * num_subcores
    b_per_w = _B // num_workers
    n_chunks = b_per_w // _C
    mesh = plsc.VectorSubcoreMesh(core_axis_name="c", subcore_axis_name="s")

    @functools.partial(
        pl.kernel,
        mesh=mesh,
        out_type=jax.ShapeDtypeStruct((_B, _D), jnp.float32),
        scratch_types=[
            pltpu.VMEM((_C,), jnp.int32),
            pltpu.VMEM((_C, _D), jnp.float32),
            pltpu.SemaphoreType.DMA,
        ],
    )
    def gather_kernel(idx_hbm, table_hbm, out_hbm, idx_v, rows_v, sem):
        wid = lax.axis_index("s") * num_cores + lax.axis_index("c")
        base = wid * b_per_w

        def body(g, carry):
            off = base + g * _C
            pltpu.sync_copy(idx_hbm.at[pl.ds(off, _C)], idx_v)
            pltpu.async_copy(table_hbm.at[idx_v], rows_v, sem).wait()
            pltpu.sync_copy(rows_v, out_hbm.at[pl.ds(off, _C)])
            return carry

        lax.fori_loop(0, n_chunks, body, 0)

    return gather_kernel


def kernel(position_ids, pos_embed):
    idx = position_ids.reshape(-1)
    out = _build_gather()(idx, pos_embed)
    return out.reshape(position_ids.shape + (pos_embed.shape[1],))


# speedup vs baseline: 2.3292x; 1.0222x over previous
import functools

import jax
import jax.numpy as jnp
from jax import lax
from jax.experimental import pallas as pl
from jax.experimental.pallas import tpu as pltpu
from jax.experimental.pallas import tpu_sc as plsc

_B = 64 * 1024
_D = 768
_C = 64


@functools.cache
def _build_gather():
    info = plsc.get_sparse_core_info()
    num_cores, num_subcores = info.num_cores, info.num_subcores
    num_workers = num_cores * num_subcores
    b_per_w = _B // num_workers
    n_chunks = b_per_w // _C
    mesh = plsc.VectorSubcoreMesh(core_axis_name="c", subcore_axis_name="s")

    @functools.partial(
        pl.kernel,
        mesh=mesh,
        out_type=jax.ShapeDtypeStruct((_B, _D), jnp.float32),
        scratch_types=[
            pltpu.VMEM((b_per_w,), jnp.int32),
            pltpu.VMEM((_C, _D), jnp.float32),
            pltpu.VMEM((_C, _D), jnp.float32),
            pltpu.SemaphoreType.DMA,
            pltpu.SemaphoreType.DMA,
        ],
    )
    def gather_kernel(idx_hbm, table_hbm, out_hbm, idx_v, rows0, rows1,
                      gsem0, gsem1):
        wid = lax.axis_index("s") * num_cores + lax.axis_index("c")
        base = wid * b_per_w
        pltpu.sync_copy(idx_hbm.at[pl.ds(base, b_per_w)], idx_v)

        rows = (rows0, rows1)
        gsem = (gsem0, gsem1)

        def start_gather(g, b):
            pltpu.async_copy(
                table_hbm.at[idx_v.at[pl.ds(g * _C, _C)]], rows[b], gsem[b])

        def wait_gather(b):
            pltpu.make_async_copy(
                table_hbm.at[pl.ds(0, _C)], rows[b], gsem[b]).wait()

        start_gather(0, 0)
        start_gather(1, 1)

        def body(i, carry):
            g0 = 2 * i
            for b in range(2):
                g = g0 + b
                wait_gather(b)
                pltpu.sync_copy(rows[b], out_hbm.at[pl.ds(base + g * _C, _C)])
                start_gather(g + 2, b)
            return carry

        lax.fori_loop(0, n_chunks // 2 - 1, body, 0)

        for b in range(2):
            g = n_chunks - 2 + b
            wait_gather(b)
            pltpu.sync_copy(rows[b], out_hbm.at[pl.ds(base + g * _C, _C)])

    return gather_kernel


def kernel(position_ids, pos_embed):
    idx = position_ids.reshape(-1)
    out = _build_gather()(idx, pos_embed)
    return out.reshape(position_ids.shape + (pos_embed.shape[1],))


# speedup vs baseline: 2.3318x; 1.0011x over previous
import functools

import jax
import jax.numpy as jnp
from jax import lax
from jax.experimental import pallas as pl
from jax.experimental.pallas import tpu as pltpu
from jax.experimental.pallas import tpu_sc as plsc

_B = 64 * 1024
_D = 768
_V = 1024
_C = 64


@functools.cache
def _build_gather():
    info = plsc.get_sparse_core_info()
    num_cores, num_subcores = info.num_cores, info.num_subcores
    num_workers = num_cores * num_subcores
    b_per_w = _B // num_workers
    n_chunks = b_per_w // _C
    mesh = plsc.VectorSubcoreMesh(core_axis_name="c", subcore_axis_name="s")

    @functools.partial(
        pl.kernel,
        mesh=mesh,
        out_type=jax.ShapeDtypeStruct((_B, _D), jnp.float32),
        scratch_types=[
            pltpu.VMEM((b_per_w,), jnp.int32),
            pltpu.VMEM((_C, _D), jnp.float32),
            pltpu.VMEM((_C, _D), jnp.float32),
            pltpu.SemaphoreType.DMA,
            pltpu.SemaphoreType.DMA,
        ],
    )
    def gather_kernel(idx_hbm, table_hbm, out_hbm, idx_v, rows0, rows1,
                      gsem0, gsem1):
        sid = lax.axis_index("s")
        wid = sid * num_cores + lax.axis_index("c")
        base = wid * b_per_w
        pltpu.sync_copy(idx_hbm.at[pl.ds(base, b_per_w)], idx_v)

        rows = (rows0, rows1)
        gsem = (gsem0, gsem1)

        def start_gather(g, b):
            pltpu.async_copy(
                table_hbm.at[idx_v.at[pl.ds(g * _C, _C)]], rows[b], gsem[b])

        def wait_gather(b):
            pltpu.make_async_copy(
                table_hbm.at[pl.ds(0, _C)], rows[b], gsem[b]).wait()

        start_gather(0, 0)
        start_gather(1, 1)

        def body(i, carry):
            g0 = 2 * i
            for b in range(2):
                g = g0 + b
                wait_gather(b)
                pltpu.sync_copy(rows[b], out_hbm.at[pl.ds(base + g * _C, _C)])
                start_gather(g + 2, b)
            return carry

        lax.fori_loop(0, n_chunks // 2 - 1, body, 0)

        for b in range(2):
            g = n_chunks - 2 + b
            wait_gather(b)
            pltpu.sync_copy(rows[b], out_hbm.at[pl.ds(base + g * _C, _C)])

    return gather_kernel


def kernel(position_ids, pos_embed):
    idx = position_ids.reshape(-1)
    out = _build_gather()(idx, pos_embed)
    return out.reshape(position_ids.shape + (pos_embed.shape[1],))
